# baseline (speedup 1.0000x reference)
import functools

import jax
import jax.numpy as jnp
from jax import lax
from jax.experimental import pallas as pl
from jax.experimental.pallas import tpu as pltpu
from jax.experimental.pallas import tpu_sc as plsc

N_NODES = 10000
IN_CH = 128
F = 16
N_EDGES = 320000
N_IDX = 2048

NC = 2
NS = 16
NW = NC * NS

NODES_PAD = 10240
SLAB = NODES_PAD // NS
CB = 128
TOT_CH = 2560
E_PAD = TOT_CH * CB
CH0 = 80
CH1 = TOT_CH // NS - CH0
IDX_PW = N_IDX // NW
NBUF = 4

_f32 = jnp.float32


def _mm1_body(x_ref, w_ref, o_ref):
    i = pl.program_id(0)
    acc = jnp.dot(x_ref[...], w_ref[...], preferred_element_type=_f32)
    rows = i * _BM + lax.broadcasted_iota(jnp.int32, (_BM, 1), 0)
    o_ref[...] = jnp.where(rows < N_NODES, acc, 0.0)


_BM = 1024

_mm1 = pl.pallas_call(
    _mm1_body,
    grid=(NODES_PAD // _BM,),
    in_specs=[
        pl.BlockSpec((_BM, IN_CH), lambda i: (i, 0)),
        pl.BlockSpec((IN_CH, F), lambda i: (0, 0)),
    ],
    out_specs=pl.BlockSpec((_BM, F), lambda i: (i, 0)),
    out_shape=jax.ShapeDtypeStruct((NODES_PAD, F), _f32),
)


_mesh = plsc.VectorSubcoreMesh(core_axis_name="c", subcore_axis_name="s")


_SC_OUT = (
    jax.ShapeDtypeStruct((NODES_PAD, F), _f32),
    jax.ShapeDtypeStruct((NODES_PAD, F), _f32),
)

_SC_SCRATCH = [
    pltpu.VMEM((CH0, CB), jnp.int32),
    pltpu.VMEM((CH0, CB), jnp.int32),
    pltpu.VMEM((NBUF, CB, F), _f32),
    pltpu.VMEM_SHARED((NODES_PAD, F), _f32),
    pltpu.VMEM_SHARED((NODES_PAD, F), _f32),
] + [pltpu.SemaphoreType.DMA] * NBUF


def _scatter_phase(ei_hbm, outa_hbm, outb_hbm, src_v, dst_v, rows_v,
                   acc_sh, tbl_sh, gsems, c, s):
    def _run(nch, base):
        pltpu.sync_copy(ei_hbm.at[0, pl.ds(base, nch)],
                        src_v.at[pl.ds(0, nch)])
        pltpu.sync_copy(ei_hbm.at[1, pl.ds(base, nch)],
                        dst_v.at[pl.ds(0, nch)])
        plsc.subcore_barrier()

        for b in range(NBUF):
            pltpu.async_copy(tbl_sh.at[src_v.at[b]], rows_v.at[b], gsems[b])

        def _group(gi, _):
            for b in range(NBUF):
                j = gi * NBUF + b
                pltpu.make_async_copy(
                    tbl_sh.at[src_v.at[j]], rows_v.at[b], gsems[b]).wait()
                pltpu.sync_copy(rows_v.at[b], acc_sh.at[dst_v.at[j]],
                                add=True)

                @pl.when(j + NBUF < nch)
                def _prefetch():
                    pltpu.async_copy(
                        tbl_sh.at[src_v.at[j + NBUF]], rows_v.at[b], gsems[b])
            return 0
        lax.fori_loop(0, nch // NBUF, _group, 0)

    @pl.when(c == 0)
    def _run0():
        _run(CH0, s * CH0)

    @pl.when(c == 1)
    def _run1():
        _run(CH1, NS * CH0 + s * CH1)

    plsc.subcore_barrier()

    @pl.when(c == 0)
    def _exa():
        pltpu.sync_copy(acc_sh.at[pl.ds(s * SLAB, SLAB)],
                        outa_hbm.at[pl.ds(s * SLAB, SLAB)])

    @pl.when(c == 1)
    def _exb():
        pltpu.sync_copy(acc_sh.at[pl.ds(s * SLAB, SLAB)],
                        outb_hbm.at[pl.ds(s * SLAB, SLAB)])


@functools.partial(
    pl.kernel,
    out_type=_SC_OUT,
    mesh=_mesh,
    scratch_types=_SC_SCRATCH,
    compiler_params=pltpu.CompilerParams(use_tc_tiling_on_sc=False),
)
def _edge_scatter1(hw_hbm, ei_hbm, zeros_hbm, outa_hbm, outb_hbm,
                   src_v, dst_v, rows_v, acc_sh, tbl_sh, *gsems):
    c = lax.axis_index("c")
    s = lax.axis_index("s")
    pltpu.sync_copy(zeros_hbm, acc_sh.at[pl.ds(s * SLAB, SLAB)])
    pltpu.sync_copy(hw_hbm.at[pl.ds(s * SLAB, SLAB)],
                    tbl_sh.at[pl.ds(s * SLAB, SLAB)])
    _scatter_phase(ei_hbm, outa_hbm, outb_hbm, src_v, dst_v, rows_v,
                   acc_sh, tbl_sh, gsems, c, s)


@functools.partial(
    pl.kernel,
    out_type=_SC_OUT,
    mesh=_mesh,
    scratch_types=_SC_SCRATCH + [
        pltpu.VMEM((SLAB, F), _f32),
        pltpu.VMEM((SLAB, F), _f32),
        pltpu.VMEM((F, F), _f32),
    ],
    compiler_params=pltpu.CompilerParams(use_tc_tiling_on_sc=False),
)
def _edge_scatter2(h1a_hbm, h1b_hbm, w2_hbm, ei_hbm, zeros_hbm,
                   outa_hbm, outb_hbm,
                   src_v, dst_v, rows_v, acc_sh, tbl_sh,
                   gs0, gs1, gs2, gs3, va_v, vb_v, w2_v):
    c = lax.axis_index("c")
    s = lax.axis_index("s")
    gsems = (gs0, gs1, gs2, gs3)
    pltpu.sync_copy(zeros_hbm, acc_sh.at[pl.ds(s * SLAB, SLAB)])
    pltpu.sync_copy(h1a_hbm.at[pl.ds(s * SLAB, SLAB)], va_v)
    pltpu.sync_copy(h1b_hbm.at[pl.ds(s * SLAB, SLAB)], vb_v)
    pltpu.sync_copy(w2_hbm, w2_v)
    w2rows = [w2_v[k] for k in range(F)]
    kvecs = [jnp.full((F,), k, jnp.int32) for k in range(F)]

    def _row(r, _):
        h = jnp.maximum(va_v[r] + vb_v[r], 0.0)
        acc = h[kvecs[0]] * w2rows[0]
        for k in range(1, F):
            acc = acc + h[kvecs[k]] * w2rows[k]
        va_v[r] = acc
        return 0
    lax.fori_loop(0, SLAB, _row, 0)
    pltpu.sync_copy(va_v, tbl_sh.at[pl.ds(s * SLAB, SLAB)])

    _scatter_phase(ei_hbm, outa_hbm, outb_hbm, src_v, dst_v, rows_v,
                   acc_sh, tbl_sh, gsems, c, s)


@functools.partial(
    pl.kernel,
    out_type=jax.ShapeDtypeStruct((N_IDX, F), _f32),
    mesh=_mesh,
    scratch_types=[
        pltpu.VMEM((IDX_PW,), jnp.int32),
        pltpu.VMEM((IDX_PW, F), _f32),
        pltpu.VMEM((IDX_PW, F), _f32),
        pltpu.VMEM((IDX_PW, F), _f32),
        pltpu.SemaphoreType.DMA,
    ],
    compiler_params=pltpu.CompilerParams(use_tc_tiling_on_sc=False),
)
def _gather_add(ha_hbm, hb_hbm, idx_hbm, out_hbm,
                idx_v, ra_v, rb_v, out_v, sem):
    c = lax.axis_index("c")
    s = lax.axis_index("s")
    wid = s * NC + c
    base = wid * IDX_PW

    pltpu.sync_copy(idx_hbm.at[pl.ds(base, IDX_PW)], idx_v)
    pltpu.async_copy(ha_hbm.at[idx_v], ra_v, sem).wait()
    pltpu.async_copy(hb_hbm.at[idx_v], rb_v, sem).wait()

    def _add(r, _):
        out_v[r] = ra_v[r] + rb_v[r]
        return 0
    lax.fori_loop(0, IDX_PW, _add, 0)

    pltpu.sync_copy(out_v, out_hbm.at[pl.ds(base, IDX_PW)])


def kernel(x, edge_index, index, W1, W2):
    pad = jnp.full((2, E_PAD - N_EDGES), N_NODES, jnp.int64)
    ei3 = jnp.concatenate([edge_index, pad], axis=1) \
             .astype(jnp.int32).reshape(2, TOT_CH, CB)
    idx32 = index.astype(jnp.int32)
    zeros_slab = jnp.zeros((SLAB, F), _f32)

    hw1 = _mm1(x, W1)
    h1a, h1b = _edge_scatter1(hw1, ei3, zeros_slab)
    h2a, h2b = _edge_scatter2(h1a, h1b, W2, ei3, zeros_slab)
    return _gather_add(h2a, h2b, idx32)


# speedup vs baseline: 1.0775x; 1.0775x over previous
import functools

import jax
import jax.numpy as jnp
from jax import lax
from jax.experimental import pallas as pl
from jax.experimental.pallas import tpu as pltpu
from jax.experimental.pallas import tpu_sc as plsc

N_NODES = 10000
IN_CH = 128
F = 16
N_EDGES = 320000
N_IDX = 2048

NC = 2
NS = 16
NW = NC * NS

NODES_PAD = 10240
SLAB = NODES_PAD // NS
CB = 128
TOT_CH = 2560
E_PAD = TOT_CH * CB
CH0 = 80
CH1 = TOT_CH // NS - CH0
IDX_PW = N_IDX // NW
NBUF = 4

_f32 = jnp.float32


def _mm1_body(x_ref, w_ref, o_ref):
    i = pl.program_id(0)
    acc = jnp.dot(x_ref[...], w_ref[...], preferred_element_type=_f32)
    rows = i * _BM + lax.broadcasted_iota(jnp.int32, (_BM, 1), 0)
    o_ref[...] = jnp.where(rows < N_NODES, acc, 0.0)


_BM = 1024

_mm1 = pl.pallas_call(
    _mm1_body,
    grid=(NODES_PAD // _BM,),
    in_specs=[
        pl.BlockSpec((_BM, IN_CH), lambda i: (i, 0)),
        pl.BlockSpec((IN_CH, F), lambda i: (0, 0)),
    ],
    out_specs=pl.BlockSpec((_BM, F), lambda i: (i, 0)),
    out_shape=jax.ShapeDtypeStruct((NODES_PAD, F), _f32),
)


_mesh = plsc.VectorSubcoreMesh(core_axis_name="c", subcore_axis_name="s")


_SC_OUT = (
    jax.ShapeDtypeStruct((NODES_PAD, F), _f32),
    jax.ShapeDtypeStruct((NODES_PAD, F), _f32),
)

_SC_SCRATCH = [
    pltpu.VMEM((CH0, CB), jnp.int32),
    pltpu.VMEM((CH0, CB), jnp.int32),
    pltpu.VMEM((NBUF, CB, F), _f32),
    pltpu.VMEM_SHARED((NODES_PAD, F), _f32),
    pltpu.VMEM_SHARED((NODES_PAD, F), _f32),
] + [pltpu.SemaphoreType.DMA] * NBUF


def _scatter_phase(ei_hbm, outa_hbm, outb_hbm, src_v, dst_v, rows_v,
                   acc_sh, tbl_sh, gsems, c, s):
    def _run(nch, base):
        pltpu.sync_copy(ei_hbm.at[0, pl.ds(base, nch)],
                        src_v.at[pl.ds(0, nch)])
        pltpu.sync_copy(ei_hbm.at[1, pl.ds(base, nch)],
                        dst_v.at[pl.ds(0, nch)])
        plsc.subcore_barrier()

        for b in range(NBUF):
            pltpu.async_copy(tbl_sh.at[src_v.at[b]], rows_v.at[b], gsems[b])

        def _group(gi, _):
            for b in range(NBUF):
                j = gi * NBUF + b
                pltpu.make_async_copy(
                    tbl_sh.at[src_v.at[j]], rows_v.at[b], gsems[b]).wait()
                pltpu.sync_copy(rows_v.at[b], acc_sh.at[dst_v.at[j]],
                                add=True)

                @pl.when(j + NBUF < nch)
                def _prefetch():
                    pltpu.async_copy(
                        tbl_sh.at[src_v.at[j + NBUF]], rows_v.at[b], gsems[b])
            return 0
        lax.fori_loop(0, nch // NBUF, _group, 0)

    @pl.when(c == 0)
    def _run0():
        _run(CH0, s * CH0)

    @pl.when(c == 1)
    def _run1():
        _run(CH1, NS * CH0 + s * CH1)

    plsc.subcore_barrier()

    @pl.when(c == 0)
    def _exa():
        pltpu.sync_copy(acc_sh.at[pl.ds(s * SLAB, SLAB)],
                        outa_hbm.at[pl.ds(s * SLAB, SLAB)])

    @pl.when(c == 1)
    def _exb():
        pltpu.sync_copy(acc_sh.at[pl.ds(s * SLAB, SLAB)],
                        outb_hbm.at[pl.ds(s * SLAB, SLAB)])


@functools.partial(
    pl.kernel,
    out_type=_SC_OUT,
    mesh=_mesh,
    scratch_types=_SC_SCRATCH,
    compiler_params=pltpu.CompilerParams(use_tc_tiling_on_sc=False),
)
def _edge_scatter1(hw_hbm, ei_hbm, zeros_hbm, outa_hbm, outb_hbm,
                   src_v, dst_v, rows_v, acc_sh, tbl_sh, *gsems):
    c = lax.axis_index("c")
    s = lax.axis_index("s")
    pltpu.sync_copy(zeros_hbm, acc_sh.at[pl.ds(s * SLAB, SLAB)])
    pltpu.sync_copy(hw_hbm.at[pl.ds(s * SLAB, SLAB)],
                    tbl_sh.at[pl.ds(s * SLAB, SLAB)])
    _scatter_phase(ei_hbm, outa_hbm, outb_hbm, src_v, dst_v, rows_v,
                   acc_sh, tbl_sh, gsems, c, s)


@functools.partial(
    pl.kernel,
    out_type=_SC_OUT,
    mesh=_mesh,
    scratch_types=_SC_SCRATCH + [
        pltpu.VMEM((SLAB, F), _f32),
        pltpu.VMEM((SLAB, F), _f32),
        pltpu.VMEM((F, F), _f32),
    ],
    compiler_params=pltpu.CompilerParams(use_tc_tiling_on_sc=False),
)
def _edge_scatter2(h1a_hbm, h1b_hbm, w2_hbm, ei_hbm, zeros_hbm,
                   outa_hbm, outb_hbm,
                   src_v, dst_v, rows_v, acc_sh, tbl_sh,
                   gs0, gs1, gs2, gs3, va_v, vb_v, w2_v):
    c = lax.axis_index("c")
    s = lax.axis_index("s")
    gsems = (gs0, gs1, gs2, gs3)
    pltpu.sync_copy(zeros_hbm, acc_sh.at[pl.ds(s * SLAB, SLAB)])
    pltpu.sync_copy(h1a_hbm.at[pl.ds(s * SLAB, SLAB)], va_v)
    pltpu.sync_copy(h1b_hbm.at[pl.ds(s * SLAB, SLAB)], vb_v)
    pltpu.sync_copy(w2_hbm, w2_v)
    w2rows = [w2_v[k] for k in range(F)]
    kvecs = [jnp.full((F,), k, jnp.int32) for k in range(F)]

    def _one(r):
        h = jnp.maximum(va_v[r] + vb_v[r], 0.0)
        parts = []
        for p in range(4):
            acc = h[kvecs[4 * p]] * w2rows[4 * p]
            for k in range(4 * p + 1, 4 * p + 4):
                acc = acc + h[kvecs[k]] * w2rows[k]
            parts.append(acc)
        va_v[r] = (parts[0] + parts[1]) + (parts[2] + parts[3])

    def _row(r4, _):
        for u in range(4):
            _one(r4 * 4 + u)
        return 0
    lax.fori_loop(0, SLAB // 4, _row, 0)
    pltpu.sync_copy(va_v, tbl_sh.at[pl.ds(s * SLAB, SLAB)])

    _scatter_phase(ei_hbm, outa_hbm, outb_hbm, src_v, dst_v, rows_v,
                   acc_sh, tbl_sh, gsems, c, s)


@functools.partial(
    pl.kernel,
    out_type=jax.ShapeDtypeStruct((N_IDX, F), _f32),
    mesh=_mesh,
    scratch_types=[
        pltpu.VMEM((IDX_PW,), jnp.int32),
        pltpu.VMEM((IDX_PW, F), _f32),
        pltpu.VMEM((IDX_PW, F), _f32),
        pltpu.VMEM((IDX_PW, F), _f32),
        pltpu.SemaphoreType.DMA,
    ],
    compiler_params=pltpu.CompilerParams(use_tc_tiling_on_sc=False),
)
def _gather_add(ha_hbm, hb_hbm, idx_hbm, out_hbm,
                idx_v, ra_v, rb_v, out_v, sem):
    c = lax.axis_index("c")
    s = lax.axis_index("s")
    wid = s * NC + c
    base = wid * IDX_PW

    pltpu.sync_copy(idx_hbm.at[pl.ds(base, IDX_PW)], idx_v)
    pltpu.async_copy(ha_hbm.at[idx_v], ra_v, sem).wait()
    pltpu.async_copy(hb_hbm.at[idx_v], rb_v, sem).wait()

    def _add(r, _):
        out_v[r] = ra_v[r] + rb_v[r]
        return 0
    lax.fori_loop(0, IDX_PW, _add, 0)

    pltpu.sync_copy(out_v, out_hbm.at[pl.ds(base, IDX_PW)])


def kernel(x, edge_index, index, W1, W2):
    pad = jnp.full((2, E_PAD - N_EDGES), N_NODES, jnp.int64)
    ei3 = jnp.concatenate([edge_index, pad], axis=1) \
             .astype(jnp.int32).reshape(2, TOT_CH, CB)
    idx32 = index.astype(jnp.int32)
    zeros_slab = jnp.zeros((SLAB, F), _f32)

    hw1 = _mm1(x, W1)
    h1a, h1b = _edge_scatter1(hw1, ei3, zeros_slab)
    h2a, h2b = _edge_scatter2(h1a, h1b, W2, ei3, zeros_slab)
    return _gather_add(h2a, h2b, idx32)


# speedup vs baseline: 1.1309x; 1.0496x over previous
import functools

import jax
import jax.numpy as jnp
from jax import lax
from jax.experimental import pallas as pl
from jax.experimental.pallas import tpu as pltpu
from jax.experimental.pallas import tpu_sc as plsc

N_NODES = 10000
IN_CH = 128
F = 16
N_EDGES = 320000
N_IDX = 2048

NC = 2
NS = 16
NW = NC * NS

NODES_PAD = 10240
SLAB = NODES_PAD // NS
CB = 128
TOT_CH = 2560
E_PAD = TOT_CH * CB
CHW = TOT_CH // NW
IDX_PW = N_IDX // NW
NBUF = 4

_f32 = jnp.float32


def _mm1_body(x_ref, w_ref, o_ref):
    i = pl.program_id(0)
    acc = jnp.dot(x_ref[...], w_ref[...], preferred_element_type=_f32)
    rows = i * _BM + lax.broadcasted_iota(jnp.int32, (_BM, 1), 0)
    o_ref[...] = jnp.where(rows < N_NODES, acc, 0.0)


_BM = 1024

_mm1 = pl.pallas_call(
    _mm1_body,
    grid=(NODES_PAD // _BM,),
    in_specs=[
        pl.BlockSpec((_BM, IN_CH), lambda i: (i, 0)),
        pl.BlockSpec((IN_CH, F), lambda i: (0, 0)),
    ],
    out_specs=pl.BlockSpec((_BM, F), lambda i: (i, 0)),
    out_shape=jax.ShapeDtypeStruct((NODES_PAD, F), _f32),
)


_mesh = plsc.VectorSubcoreMesh(core_axis_name="c", subcore_axis_name="s")


_SC_OUT = (
    jax.ShapeDtypeStruct((NODES_PAD, F), _f32),
    jax.ShapeDtypeStruct((NODES_PAD, F), _f32),
)

_SC_SCRATCH = [
    pltpu.VMEM((CHW, CB), jnp.int32),
    pltpu.VMEM((CHW, CB), jnp.int32),
    pltpu.VMEM((NBUF, CB, F), _f32),
    pltpu.VMEM_SHARED((NODES_PAD, F), _f32),
    pltpu.VMEM_SHARED((NODES_PAD, F), _f32),
] + [pltpu.SemaphoreType.DMA] * (NBUF + 2)


def _scatter_phase(outa_hbm, outb_hbm, src_v, dst_v, rows_v,
                   acc_sh, tbl_sh, gsems, c, s):
    plsc.subcore_barrier()

    for b in range(NBUF):
        pltpu.async_copy(tbl_sh.at[src_v.at[b]], rows_v.at[b], gsems[b])

    def _group(gi, _):
        for b in range(NBUF):
            j = gi * NBUF + b
            pltpu.make_async_copy(
                tbl_sh.at[src_v.at[j]], rows_v.at[b], gsems[b]).wait()
            pltpu.sync_copy(rows_v.at[b], acc_sh.at[dst_v.at[j]],
                            add=True)

            @pl.when(j + NBUF < CHW)
            def _prefetch():
                pltpu.async_copy(
                    tbl_sh.at[src_v.at[j + NBUF]], rows_v.at[b], gsems[b])
        return 0
    lax.fori_loop(0, CHW // NBUF, _group, 0)
    plsc.subcore_barrier()

    @pl.when(c == 0)
    def _exa():
        pltpu.sync_copy(acc_sh.at[pl.ds(s * SLAB, SLAB)],
                        outa_hbm.at[pl.ds(s * SLAB, SLAB)])

    @pl.when(c == 1)
    def _exb():
        pltpu.sync_copy(acc_sh.at[pl.ds(s * SLAB, SLAB)],
                        outb_hbm.at[pl.ds(s * SLAB, SLAB)])


@functools.partial(
    pl.kernel,
    out_type=_SC_OUT,
    mesh=_mesh,
    scratch_types=_SC_SCRATCH,
    compiler_params=pltpu.CompilerParams(use_tc_tiling_on_sc=False),
)
def _edge_scatter1(hw_hbm, ei_hbm, zeros_hbm, outa_hbm, outb_hbm,
                   src_v, dst_v, rows_v, acc_sh, tbl_sh, *gsems):
    c = lax.axis_index("c")
    s = lax.axis_index("s")
    base = (c * NS + s) * CHW
    dz = pltpu.async_copy(zeros_hbm, acc_sh.at[pl.ds(s * SLAB, SLAB)],
                          gsems[0])
    dt = pltpu.async_copy(hw_hbm.at[pl.ds(s * SLAB, SLAB)],
                          tbl_sh.at[pl.ds(s * SLAB, SLAB)], gsems[1])
    ds_ = pltpu.async_copy(ei_hbm.at[0, pl.ds(base, CHW)], src_v, gsems[2])
    dd = pltpu.async_copy(ei_hbm.at[1, pl.ds(base, CHW)], dst_v, gsems[3])
    dz.wait(); dt.wait(); ds_.wait(); dd.wait()
    _scatter_phase(outa_hbm, outb_hbm, src_v, dst_v, rows_v,
                   acc_sh, tbl_sh, gsems, c, s)


@functools.partial(
    pl.kernel,
    out_type=_SC_OUT,
    mesh=_mesh,
    scratch_types=_SC_SCRATCH + [
        pltpu.VMEM((SLAB, F), _f32),
        pltpu.VMEM((SLAB, F), _f32),
        pltpu.VMEM((F, F), _f32),
    ],
    compiler_params=pltpu.CompilerParams(use_tc_tiling_on_sc=False),
)
def _edge_scatter2(h1a_hbm, h1b_hbm, w2_hbm, ei_hbm, zeros_hbm,
                   outa_hbm, outb_hbm,
                   src_v, dst_v, rows_v, acc_sh, tbl_sh,
                   gs0, gs1, gs2, gs3, gs4, gs5, va_v, vb_v, w2_v):
    c = lax.axis_index("c")
    s = lax.axis_index("s")
    gsems = (gs0, gs1, gs2, gs3)
    base = (c * NS + s) * CHW
    da = pltpu.async_copy(h1a_hbm.at[pl.ds(s * SLAB, SLAB)], va_v, gs0)
    db = pltpu.async_copy(h1b_hbm.at[pl.ds(s * SLAB, SLAB)], vb_v, gs1)
    dw = pltpu.async_copy(w2_hbm, w2_v, gs2)
    dz = pltpu.async_copy(zeros_hbm, acc_sh.at[pl.ds(s * SLAB, SLAB)], gs3)
    ds_ = pltpu.async_copy(ei_hbm.at[0, pl.ds(base, CHW)], src_v, gs4)
    dd = pltpu.async_copy(ei_hbm.at[1, pl.ds(base, CHW)], dst_v, gs5)
    da.wait(); db.wait(); dw.wait()

    w2rows = [w2_v[k] for k in range(F)]
    kvecs = [jnp.full((F,), k, jnp.int32) for k in range(F)]

    def _one(r):
        h = jnp.maximum(va_v[r] + vb_v[r], 0.0)
        parts = []
        for p in range(4):
            acc = h[kvecs[4 * p]] * w2rows[4 * p]
            for k in range(4 * p + 1, 4 * p + 4):
                acc = acc + h[kvecs[k]] * w2rows[k]
            parts.append(acc)
        va_v[r] = (parts[0] + parts[1]) + (parts[2] + parts[3])

    def _row(r4, _):
        for u in range(4):
            _one(r4 * 4 + u)
        return 0
    lax.fori_loop(0, SLAB // 4, _row, 0)
    pltpu.sync_copy(va_v, tbl_sh.at[pl.ds(s * SLAB, SLAB)])
    dz.wait(); ds_.wait(); dd.wait()

    _scatter_phase(outa_hbm, outb_hbm, src_v, dst_v, rows_v,
                   acc_sh, tbl_sh, gsems, c, s)


@functools.partial(
    pl.kernel,
    out_type=jax.ShapeDtypeStruct((N_IDX, F), _f32),
    mesh=_mesh,
    scratch_types=[
        pltpu.VMEM((IDX_PW,), jnp.int32),
        pltpu.VMEM((IDX_PW, F), _f32),
        pltpu.VMEM((IDX_PW, F), _f32),
        pltpu.VMEM((IDX_PW, F), _f32),
        pltpu.SemaphoreType.DMA,
    ],
    compiler_params=pltpu.CompilerParams(use_tc_tiling_on_sc=False),
)
def _gather_add(ha_hbm, hb_hbm, idx_hbm, out_hbm,
                idx_v, ra_v, rb_v, out_v, sem):
    c = lax.axis_index("c")
    s = lax.axis_index("s")
    wid = s * NC + c
    base = wid * IDX_PW

    pltpu.sync_copy(idx_hbm.at[pl.ds(base, IDX_PW)], idx_v)
    pltpu.async_copy(ha_hbm.at[idx_v], ra_v, sem).wait()
    pltpu.async_copy(hb_hbm.at[idx_v], rb_v, sem).wait()

    def _add(r, _):
        out_v[r] = ra_v[r] + rb_v[r]
        return 0
    lax.fori_loop(0, IDX_PW, _add, 0)

    pltpu.sync_copy(out_v, out_hbm.at[pl.ds(base, IDX_PW)])


def kernel(x, edge_index, index, W1, W2):
    pad = jnp.full((2, E_PAD - N_EDGES), N_NODES, jnp.int64)
    ei3 = jnp.concatenate([edge_index, pad], axis=1) \
             .astype(jnp.int32).reshape(2, TOT_CH, CB)
    idx32 = index.astype(jnp.int32)
    zeros_slab = jnp.zeros((SLAB, F), _f32)

    hw1 = _mm1(x, W1)
    h1a, h1b = _edge_scatter1(hw1, ei3, zeros_slab)
    h2a, h2b = _edge_scatter2(h1a, h1b, W2, ei3, zeros_slab)
    return _gather_add(h2a, h2b, idx32)


# speedup vs baseline: 1.1314x; 1.0004x over previous
import functools

import jax
import jax.numpy as jnp
from jax import lax
from jax.experimental import pallas as pl
from jax.experimental.pallas import tpu as pltpu
from jax.experimental.pallas import tpu_sc as plsc

N_NODES = 10000
IN_CH = 128
F = 16
N_EDGES = 320000
N_IDX = 2048

NC = 2
NS = 16
NW = NC * NS

NODES_PAD = 10240
SLAB = NODES_PAD // NS
CB = 128
TOT_CH = 2560
E_PAD = TOT_CH * CB
CHW = TOT_CH // NW
IDX_PW = N_IDX // NW
NBUF = 8

_f32 = jnp.float32


def _mm1_body(x_ref, w_ref, o_ref):
    i = pl.program_id(0)
    acc = jnp.dot(x_ref[...], w_ref[...], preferred_element_type=_f32)
    rows = i * _BM + lax.broadcasted_iota(jnp.int32, (_BM, 1), 0)
    o_ref[...] = jnp.where(rows < N_NODES, acc, 0.0)


_BM = 1024

_mm1 = pl.pallas_call(
    _mm1_body,
    grid=(NODES_PAD // _BM,),
    in_specs=[
        pl.BlockSpec((_BM, IN_CH), lambda i: (i, 0)),
        pl.BlockSpec((IN_CH, F), lambda i: (0, 0)),
    ],
    out_specs=pl.BlockSpec((_BM, F), lambda i: (i, 0)),
    out_shape=jax.ShapeDtypeStruct((NODES_PAD, F), _f32),
)


_mesh = plsc.VectorSubcoreMesh(core_axis_name="c", subcore_axis_name="s")


_SC_OUT = (
    jax.ShapeDtypeStruct((NODES_PAD, F), _f32),
    jax.ShapeDtypeStruct((NODES_PAD, F), _f32),
)

_SC_SCRATCH = [
    pltpu.VMEM((CHW, CB), jnp.int32),
    pltpu.VMEM((CHW, CB), jnp.int32),
    pltpu.VMEM((NBUF, CB, F), _f32),
    pltpu.VMEM_SHARED((NODES_PAD, F), _f32),
    pltpu.VMEM_SHARED((NODES_PAD, F), _f32),
] + [pltpu.SemaphoreType.DMA] * (NBUF + 2)


def _scatter_phase(outa_hbm, outb_hbm, src_v, dst_v, rows_v,
                   acc_sh, tbl_sh, gsems, c, s):
    plsc.subcore_barrier()

    for b in range(NBUF):
        pltpu.async_copy(tbl_sh.at[src_v.at[b]], rows_v.at[b], gsems[b])

    def _group(gi, _):
        for b in range(NBUF):
            j = gi * NBUF + b
            pltpu.make_async_copy(
                tbl_sh.at[src_v.at[j]], rows_v.at[b], gsems[b]).wait()
            pltpu.sync_copy(rows_v.at[b], acc_sh.at[dst_v.at[j]],
                            add=True)

            @pl.when(j + NBUF < CHW)
            def _prefetch():
                pltpu.async_copy(
                    tbl_sh.at[src_v.at[j + NBUF]], rows_v.at[b], gsems[b])
        return 0
    lax.fori_loop(0, CHW // NBUF, _group, 0)
    plsc.subcore_barrier()

    @pl.when(c == 0)
    def _exa():
        pltpu.sync_copy(acc_sh.at[pl.ds(s * SLAB, SLAB)],
                        outa_hbm.at[pl.ds(s * SLAB, SLAB)])

    @pl.when(c == 1)
    def _exb():
        pltpu.sync_copy(acc_sh.at[pl.ds(s * SLAB, SLAB)],
                        outb_hbm.at[pl.ds(s * SLAB, SLAB)])


@functools.partial(
    pl.kernel,
    out_type=_SC_OUT,
    mesh=_mesh,
    scratch_types=_SC_SCRATCH,
    compiler_params=pltpu.CompilerParams(use_tc_tiling_on_sc=False),
)
def _edge_scatter1(hw_hbm, ei_hbm, zeros_hbm, outa_hbm, outb_hbm,
                   src_v, dst_v, rows_v, acc_sh, tbl_sh, *gsems):
    c = lax.axis_index("c")
    s = lax.axis_index("s")
    base = (c * NS + s) * CHW
    dz = pltpu.async_copy(zeros_hbm, acc_sh.at[pl.ds(s * SLAB, SLAB)],
                          gsems[0])
    dt = pltpu.async_copy(hw_hbm.at[pl.ds(s * SLAB, SLAB)],
                          tbl_sh.at[pl.ds(s * SLAB, SLAB)], gsems[1])
    ds_ = pltpu.async_copy(ei_hbm.at[0, pl.ds(base, CHW)], src_v, gsems[2])
    dd = pltpu.async_copy(ei_hbm.at[1, pl.ds(base, CHW)], dst_v, gsems[3])
    dz.wait(); dt.wait(); ds_.wait(); dd.wait()
    _scatter_phase(outa_hbm, outb_hbm, src_v, dst_v, rows_v,
                   acc_sh, tbl_sh, gsems, c, s)


@functools.partial(
    pl.kernel,
    out_type=_SC_OUT,
    mesh=_mesh,
    scratch_types=_SC_SCRATCH + [
        pltpu.VMEM((SLAB, F), _f32),
        pltpu.VMEM((SLAB, F), _f32),
        pltpu.VMEM((F, F), _f32),
    ],
    compiler_params=pltpu.CompilerParams(use_tc_tiling_on_sc=False),
)
def _edge_scatter2(h1a_hbm, h1b_hbm, w2_hbm, ei_hbm, zeros_hbm,
                   outa_hbm, outb_hbm,
                   src_v, dst_v, rows_v, acc_sh, tbl_sh, *rest):
    gsems = rest[:NBUF + 2]
    va_v, vb_v, w2_v = rest[NBUF + 2:]
    c = lax.axis_index("c")
    s = lax.axis_index("s")
    base = (c * NS + s) * CHW
    da = pltpu.async_copy(h1a_hbm.at[pl.ds(s * SLAB, SLAB)], va_v, gsems[0])
    db = pltpu.async_copy(h1b_hbm.at[pl.ds(s * SLAB, SLAB)], vb_v, gsems[1])
    dw = pltpu.async_copy(w2_hbm, w2_v, gsems[2])
    dz = pltpu.async_copy(zeros_hbm, acc_sh.at[pl.ds(s * SLAB, SLAB)],
                          gsems[3])
    ds_ = pltpu.async_copy(ei_hbm.at[0, pl.ds(base, CHW)], src_v, gsems[4])
    dd = pltpu.async_copy(ei_hbm.at[1, pl.ds(base, CHW)], dst_v, gsems[5])
    da.wait(); db.wait(); dw.wait()

    w2rows = [w2_v[k] for k in range(F)]
    kvecs = [jnp.full((F,), k, jnp.int32) for k in range(F)]

    def _one(r):
        h = jnp.maximum(va_v[r] + vb_v[r], 0.0)
        parts = []
        for p in range(4):
            acc = h[kvecs[4 * p]] * w2rows[4 * p]
            for k in range(4 * p + 1, 4 * p + 4):
                acc = acc + h[kvecs[k]] * w2rows[k]
            parts.append(acc)
        va_v[r] = (parts[0] + parts[1]) + (parts[2] + parts[3])

    def _row(r4, _):
        for u in range(4):
            _one(r4 * 4 + u)
        return 0
    lax.fori_loop(0, SLAB // 4, _row, 0)
    pltpu.sync_copy(va_v, tbl_sh.at[pl.ds(s * SLAB, SLAB)])
    dz.wait(); ds_.wait(); dd.wait()

    _scatter_phase(outa_hbm, outb_hbm, src_v, dst_v, rows_v,
                   acc_sh, tbl_sh, gsems, c, s)


@functools.partial(
    pl.kernel,
    out_type=jax.ShapeDtypeStruct((N_IDX, F), _f32),
    mesh=_mesh,
    scratch_types=[
        pltpu.VMEM((IDX_PW,), jnp.int32),
        pltpu.VMEM((IDX_PW, F), _f32),
        pltpu.VMEM((IDX_PW, F), _f32),
        pltpu.VMEM((IDX_PW, F), _f32),
        pltpu.SemaphoreType.DMA,
    ],
    compiler_params=pltpu.CompilerParams(use_tc_tiling_on_sc=False),
)
def _gather_add(ha_hbm, hb_hbm, idx_hbm, out_hbm,
                idx_v, ra_v, rb_v, out_v, sem):
    c = lax.axis_index("c")
    s = lax.axis_index("s")
    wid = s * NC + c
    base = wid * IDX_PW

    pltpu.sync_copy(idx_hbm.at[pl.ds(base, IDX_PW)], idx_v)
    pltpu.async_copy(ha_hbm.at[idx_v], ra_v, sem).wait()
    pltpu.async_copy(hb_hbm.at[idx_v], rb_v, sem).wait()

    def _add(r, _):
        out_v[r] = ra_v[r] + rb_v[r]
        return 0
    lax.fori_loop(0, IDX_PW, _add, 0)

    pltpu.sync_copy(out_v, out_hbm.at[pl.ds(base, IDX_PW)])


def kernel(x, edge_index, index, W1, W2):
    pad = jnp.full((2, E_PAD - N_EDGES), N_NODES, jnp.int64)
    ei3 = jnp.concatenate([edge_index, pad], axis=1) \
             .astype(jnp.int32).reshape(2, TOT_CH, CB)
    idx32 = index.astype(jnp.int32)
    zeros_slab = jnp.zeros((SLAB, F), _f32)

    hw1 = _mm1(x, W1)
    h1a, h1b = _edge_scatter1(hw1, ei3, zeros_slab)
    h2a, h2b = _edge_scatter2(h1a, h1b, W2, ei3, zeros_slab)
    return _gather_add(h2a, h2b, idx32)
